# Initial kernel scaffold; baseline (speedup 1.0000x reference)
import functools

import jax
import jax.numpy as jnp
from jax import lax
from jax.experimental import pallas as pl
from jax.experimental.pallas import tpu as pltpu
from jax.experimental.pallas import tpu_sc as plsc

N_NODES = 10000
N_EDGES = 320000
D = 128
EF = 16

NUM_CORES = 2
NUM_SUBCORES = 16
NUM_TILES = NUM_CORES * NUM_SUBCORES
CHUNK = 128
CHUNKS_PER_TILE = 79
E_PER_TILE = CHUNK * CHUNKS_PER_TILE
E_PAD = NUM_TILES * E_PER_TILE
ACC_ROWS = 10112
ROWS_PER_TILE = N_NODES // NUM_SUBCORES


def _mm_bias_body(x_ref, w_ref, b_ref, o_ref):
    o_ref[...] = (
        jnp.dot(x_ref[...], w_ref[...], preferred_element_type=jnp.float32)
        + b_ref[...]
    )


def _mm_body(x_ref, w_ref, o_ref):
    o_ref[...] = jnp.dot(x_ref[...], w_ref[...], preferred_element_type=jnp.float32)


def _apply_body(x_ref, p0_ref, p1_ref, w1_ref, w2_ref, b_ref, o_ref):
    m = p0_ref[...] + p1_ref[...]
    acc = jnp.dot(x_ref[...], w1_ref[...], preferred_element_type=jnp.float32)
    acc = acc + jnp.dot(m, w2_ref[...], preferred_element_type=jnp.float32)
    o_ref[...] = jnp.maximum(acc + b_ref[...], 0.0)


_sc_mesh = plsc.VectorSubcoreMesh(core_axis_name="c", subcore_axis_name="s")


@functools.partial(
    pl.kernel,
    out_type=jax.ShapeDtypeStruct((NUM_CORES * N_NODES, D), jnp.float32),
    mesh=_sc_mesh,
    scratch_types=[
        pltpu.VMEM((CHUNK,), jnp.int32),
        pltpu.VMEM((CHUNK,), jnp.int32),
        pltpu.VMEM((CHUNK, D), jnp.float32),
        pltpu.VMEM((CHUNK, D), jnp.float32),
        pltpu.VMEM_SHARED((ACC_ROWS, D), jnp.float32),
        pltpu.SemaphoreType.DMA,
    ],
)
def _sc_gather_scatter(a_hbm, b_hbm, src_hbm, dst_hbm, out_hbm,
                       src_v, dst_v, a_v, b_v, acc, sem):
    cid = lax.axis_index("c")
    sid = lax.axis_index("s")
    wid = cid * NUM_SUBCORES + sid

    zero = jnp.zeros((16,), jnp.float32)

    def zero_row(r, carry):
        for j in range(8):
            b_v[r, pl.ds(j * 16, 16)] = zero
        return carry

    lax.fori_loop(0, CHUNK, zero_row, 0)
    zbase = sid * ROWS_PER_TILE
    for k in range(ROWS_PER_TILE // CHUNK):
        pltpu.sync_copy(b_v, acc.at[pl.ds(zbase + k * CHUNK, CHUNK)])
    rem = ROWS_PER_TILE % CHUNK
    if rem:
        pltpu.sync_copy(
            b_v.at[pl.ds(0, rem)],
            acc.at[pl.ds(zbase + (ROWS_PER_TILE // CHUNK) * CHUNK, rem)],
        )
    plsc.subcore_barrier()

    ebase = wid * E_PER_TILE

    def chunk_body(c, carry):
        base = ebase + c * CHUNK
        pltpu.sync_copy(src_hbm.at[pl.ds(base, CHUNK)], src_v)
        pltpu.sync_copy(dst_hbm.at[pl.ds(base, CHUNK)], dst_v)
        pltpu.async_copy(a_hbm.at[src_v], a_v, sem).wait()
        pltpu.sync_copy(b_hbm.at[pl.ds(base, CHUNK)], b_v)

        def msg_row(r, inner):
            for j in range(8):
                s = pl.ds(j * 16, 16)
                a_v[r, s] = jnp.maximum(a_v[r, s] + b_v[r, s], 0.0)
            return inner

        lax.fori_loop(0, CHUNK, msg_row, 0)
        pltpu.sync_copy(a_v, acc.at[dst_v], add=True)
        return carry

    lax.fori_loop(0, CHUNKS_PER_TILE, chunk_body, 0)
    plsc.subcore_barrier()

    obase = sid * ROWS_PER_TILE
    pltpu.sync_copy(
        acc.at[pl.ds(obase, ROWS_PER_TILE)],
        out_hbm.at[pl.ds(cid * N_NODES + obase, ROWS_PER_TILE)],
    )


def kernel(node_feats, edge_index, edge_feats, W_msg, b_msg, W_apply, b_apply):
    src = edge_index[0].astype(jnp.int32)
    dst = edge_index[1].astype(jnp.int32)
    pad = E_PAD - N_EDGES
    src = jnp.concatenate([src, jnp.zeros((pad,), jnp.int32)])
    dst = jnp.concatenate([dst, jnp.full((pad,), N_NODES, jnp.int32)])
    ef_pad = jnp.concatenate(
        [edge_feats, jnp.zeros((pad, EF), jnp.float32)], axis=0
    )

    w1t = W_msg[:, :D].T
    w2t = W_msg[:, D:].T
    wa1t = W_apply[:, :D].T
    wa2t = W_apply[:, D:].T
    bm = b_msg.reshape(1, D)
    ba = b_apply.reshape(1, D)

    blk_a = 1000
    a = pl.pallas_call(
        _mm_bias_body,
        grid=(N_NODES // blk_a,),
        in_specs=[
            pl.BlockSpec((blk_a, D), lambda i: (i, 0)),
            pl.BlockSpec((D, D), lambda i: (0, 0)),
            pl.BlockSpec((1, D), lambda i: (0, 0)),
        ],
        out_specs=pl.BlockSpec((blk_a, D), lambda i: (i, 0)),
        out_shape=jax.ShapeDtypeStruct((N_NODES, D), jnp.float32),
    )(node_feats, w1t, bm)

    blk_b = 1024
    b = pl.pallas_call(
        _mm_body,
        grid=(E_PAD // blk_b,),
        in_specs=[
            pl.BlockSpec((blk_b, EF), lambda i: (i, 0)),
            pl.BlockSpec((EF, D), lambda i: (0, 0)),
        ],
        out_specs=pl.BlockSpec((blk_b, D), lambda i: (i, 0)),
        out_shape=jax.ShapeDtypeStruct((E_PAD, D), jnp.float32),
    )(ef_pad, w2t)

    partials = _sc_gather_scatter(a, b, src, dst)
    p0 = partials[:N_NODES]
    p1 = partials[N_NODES:]

    out = pl.pallas_call(
        _apply_body,
        grid=(N_NODES // blk_a,),
        in_specs=[
            pl.BlockSpec((blk_a, D), lambda i: (i, 0)),
            pl.BlockSpec((blk_a, D), lambda i: (i, 0)),
            pl.BlockSpec((blk_a, D), lambda i: (i, 0)),
            pl.BlockSpec((D, D), lambda i: (0, 0)),
            pl.BlockSpec((D, D), lambda i: (0, 0)),
            pl.BlockSpec((1, D), lambda i: (0, 0)),
        ],
        out_specs=pl.BlockSpec((blk_a, D), lambda i: (i, 0)),
        out_shape=jax.ShapeDtypeStruct((N_NODES, D), jnp.float32),
    )(node_feats, p0, p1, wa1t, wa2t, ba)
    return out


# speedup vs baseline: 2.0926x; 2.0926x over previous
import functools

import jax
import jax.numpy as jnp
from jax import lax
from jax.experimental import pallas as pl
from jax.experimental.pallas import tpu as pltpu
from jax.experimental.pallas import tpu_sc as plsc

N_NODES = 10000
N_EDGES = 320000
D = 128
EF = 16

NUM_CORES = 2
NUM_SUBCORES = 16
NUM_TILES = NUM_CORES * NUM_SUBCORES
CHUNK = 128
CHUNKS_PER_TILE = 79
E_PER_TILE = CHUNK * CHUNKS_PER_TILE
E_PAD = NUM_TILES * E_PER_TILE
ACC_ROWS = 10240
ROWS_PER_TILE = ACC_ROWS // NUM_SUBCORES


def _mm_bias_body(x_ref, w_ref, b_ref, o_ref):
    o_ref[...] = (
        jnp.dot(x_ref[...], w_ref[...], preferred_element_type=jnp.float32)
        + b_ref[...]
    )


def _mm_body(x_ref, w_ref, o_ref):
    o_ref[...] = jnp.dot(x_ref[...], w_ref[...], preferred_element_type=jnp.float32)


def _apply_body(x_ref, p0_ref, p1_ref, w1_ref, w2_ref, b_ref, o_ref):
    m = p0_ref[...] + p1_ref[...]
    acc = jnp.dot(x_ref[...], w1_ref[...], preferred_element_type=jnp.float32)
    acc = acc + jnp.dot(m, w2_ref[...], preferred_element_type=jnp.float32)
    o_ref[...] = jnp.maximum(acc + b_ref[...], 0.0)


_sc_mesh = plsc.VectorSubcoreMesh(core_axis_name="c", subcore_axis_name="s")


@functools.partial(
    pl.kernel,
    out_type=jax.ShapeDtypeStruct((NUM_CORES * ACC_ROWS, D), jnp.float32),
    mesh=_sc_mesh,
    scratch_types=[
        pltpu.VMEM((CHUNK,), jnp.int32),
        pltpu.VMEM((CHUNK,), jnp.int32),
        pltpu.VMEM((CHUNK, D), jnp.float32),
        pltpu.VMEM((CHUNK, D), jnp.float32),
        pltpu.VMEM_SHARED((ACC_ROWS, D), jnp.float32),
        pltpu.SemaphoreType.DMA,
    ],
)
def _sc_gather_scatter(a_hbm, b_hbm, src_hbm, dst_hbm, out_hbm,
                       src_v, dst_v, a_v, b_v, acc, sem):
    cid = lax.axis_index("c")
    sid = lax.axis_index("s")
    wid = cid * NUM_SUBCORES + sid

    zero = jnp.zeros((16,), jnp.float32)

    def zero_row(r, carry):
        for j in range(8):
            b_v[r, pl.ds(j * 16, 16)] = zero
        return carry

    lax.fori_loop(0, CHUNK, zero_row, 0)
    zbase = sid * ROWS_PER_TILE
    for k in range(ROWS_PER_TILE // CHUNK):
        pltpu.sync_copy(b_v, acc.at[pl.ds(zbase + k * CHUNK, CHUNK)])
    plsc.subcore_barrier()

    ebase = wid * E_PER_TILE

    def chunk_body(c, carry):
        base = ebase + c * CHUNK
        pltpu.sync_copy(src_hbm.at[pl.ds(base, CHUNK)], src_v)
        pltpu.sync_copy(dst_hbm.at[pl.ds(base, CHUNK)], dst_v)
        pltpu.async_copy(a_hbm.at[src_v], a_v, sem).wait()
        pltpu.sync_copy(b_hbm.at[pl.ds(base, CHUNK)], b_v)

        def msg_row(r, inner):
            for j in range(8):
                s = pl.ds(j * 16, 16)
                a_v[r, s] = jnp.maximum(a_v[r, s] + b_v[r, s], 0.0)
            return inner

        lax.fori_loop(0, CHUNK, msg_row, 0)
        pltpu.sync_copy(a_v, acc.at[dst_v], add=True)
        return carry

    lax.fori_loop(0, CHUNKS_PER_TILE, chunk_body, 0)
    plsc.subcore_barrier()

    obase = sid * ROWS_PER_TILE
    pltpu.sync_copy(
        acc.at[pl.ds(obase, ROWS_PER_TILE)],
        out_hbm.at[pl.ds(cid * ACC_ROWS + obase, ROWS_PER_TILE)],
    )


def kernel(node_feats, edge_index, edge_feats, W_msg, b_msg, W_apply, b_apply):
    src = edge_index[0].astype(jnp.int32)
    dst = edge_index[1].astype(jnp.int32)
    pad = E_PAD - N_EDGES
    src = jnp.concatenate([src, jnp.zeros((pad,), jnp.int32)])
    dst = jnp.concatenate([dst, jnp.full((pad,), N_NODES, jnp.int32)])
    ef_pad = jnp.concatenate(
        [edge_feats, jnp.zeros((pad, EF), jnp.float32)], axis=0
    )

    w1t = W_msg[:, :D].T
    w2t = W_msg[:, D:].T
    wa1t = W_apply[:, :D].T
    wa2t = W_apply[:, D:].T
    bm = b_msg.reshape(1, D)
    ba = b_apply.reshape(1, D)

    blk_a = 1000
    a = pl.pallas_call(
        _mm_bias_body,
        grid=(N_NODES // blk_a,),
        in_specs=[
            pl.BlockSpec((blk_a, D), lambda i: (i, 0)),
            pl.BlockSpec((D, D), lambda i: (0, 0)),
            pl.BlockSpec((1, D), lambda i: (0, 0)),
        ],
        out_specs=pl.BlockSpec((blk_a, D), lambda i: (i, 0)),
        out_shape=jax.ShapeDtypeStruct((N_NODES, D), jnp.float32),
    )(node_feats, w1t, bm)

    blk_b = 1024
    b = pl.pallas_call(
        _mm_body,
        grid=(E_PAD // blk_b,),
        in_specs=[
            pl.BlockSpec((blk_b, EF), lambda i: (i, 0)),
            pl.BlockSpec((EF, D), lambda i: (0, 0)),
        ],
        out_specs=pl.BlockSpec((blk_b, D), lambda i: (i, 0)),
        out_shape=jax.ShapeDtypeStruct((E_PAD, D), jnp.float32),
    )(ef_pad, w2t)

    partials = _sc_gather_scatter(a, b, src, dst)
    p0 = partials[:N_NODES]
    p1 = partials[ACC_ROWS:ACC_ROWS + N_NODES]

    out = pl.pallas_call(
        _apply_body,
        grid=(N_NODES // blk_a,),
        in_specs=[
            pl.BlockSpec((blk_a, D), lambda i: (i, 0)),
            pl.BlockSpec((blk_a, D), lambda i: (i, 0)),
            pl.BlockSpec((blk_a, D), lambda i: (i, 0)),
            pl.BlockSpec((D, D), lambda i: (0, 0)),
            pl.BlockSpec((D, D), lambda i: (0, 0)),
            pl.BlockSpec((1, D), lambda i: (0, 0)),
        ],
        out_specs=pl.BlockSpec((blk_a, D), lambda i: (i, 0)),
        out_shape=jax.ShapeDtypeStruct((N_NODES, D), jnp.float32),
    )(node_feats, p0, p1, wa1t, wa2t, ba)
    return out


# speedup vs baseline: 2.2910x; 1.0948x over previous
import functools

import jax
import jax.numpy as jnp
from jax import lax
from jax.experimental import pallas as pl
from jax.experimental.pallas import tpu as pltpu
from jax.experimental.pallas import tpu_sc as plsc

N_NODES = 10000
N_EDGES = 320000
D = 128
EF = 16

NUM_CORES = 2
NUM_SUBCORES = 16
NUM_TILES = NUM_CORES * NUM_SUBCORES
CHUNK = 64
CHUNKS_PER_TILE = 160
E_PER_TILE = CHUNK * CHUNKS_PER_TILE
E_PAD = NUM_TILES * E_PER_TILE
ACC_ROWS = 10240
ROWS_PER_TILE = ACC_ROWS // NUM_SUBCORES


def _mm_bias_body(x_ref, w_ref, b_ref, o_ref):
    o_ref[...] = (
        jnp.dot(x_ref[...], w_ref[...], preferred_element_type=jnp.float32)
        + b_ref[...]
    )


def _mm_body(x_ref, w_ref, o_ref):
    o_ref[...] = jnp.dot(x_ref[...], w_ref[...], preferred_element_type=jnp.float32)


def _apply_body(x_ref, p0_ref, p1_ref, w1_ref, w2_ref, b_ref, o_ref):
    m = p0_ref[...] + p1_ref[...]
    acc = jnp.dot(x_ref[...], w1_ref[...], preferred_element_type=jnp.float32)
    acc = acc + jnp.dot(m, w2_ref[...], preferred_element_type=jnp.float32)
    o_ref[...] = jnp.maximum(acc + b_ref[...], 0.0)


_sc_mesh = plsc.VectorSubcoreMesh(core_axis_name="c", subcore_axis_name="s")


@functools.partial(
    pl.kernel,
    out_type=[
        jax.ShapeDtypeStruct((ACC_ROWS, D), jnp.float32),
        jax.ShapeDtypeStruct((ACC_ROWS, D), jnp.float32),
    ],
    mesh=_sc_mesh,
    scratch_types=[
        pltpu.VMEM((CHUNK,), jnp.int32),
        pltpu.VMEM((CHUNK,), jnp.int32),
        pltpu.VMEM((CHUNK,), jnp.int32),
        pltpu.VMEM((CHUNK,), jnp.int32),
        pltpu.VMEM((CHUNK,), jnp.int32),
        pltpu.VMEM((CHUNK,), jnp.int32),
        pltpu.VMEM((CHUNK, D), jnp.float32),
        pltpu.VMEM((CHUNK, D), jnp.float32),
        pltpu.VMEM((CHUNK, D), jnp.float32),
        pltpu.VMEM((CHUNK, D), jnp.float32),
        pltpu.VMEM_SHARED((ACC_ROWS, D), jnp.float32),
        pltpu.SemaphoreType.DMA,
        pltpu.SemaphoreType.DMA,
        pltpu.SemaphoreType.DMA,
        pltpu.SemaphoreType.DMA,
        pltpu.SemaphoreType.DMA,
        pltpu.SemaphoreType.DMA,
    ],
)
def _sc_gather_scatter(a_hbm, b_hbm, idx_hbm, out0_hbm, out1_hbm,
                       pk_s0, pk_s1, src_s0, dst_s0, src_s1, dst_s1,
                       a_v0, b_v0, a_v1, b_v1, acc,
                       sem_i0, sem_i1, sem_a0, sem_b0, sem_a1, sem_b1):
    cid = lax.axis_index("c")
    sid = lax.axis_index("s")
    wid = cid * NUM_SUBCORES + sid

    zero = jnp.zeros((16,), jnp.float32)

    @plsc.parallel_loop(0, CHUNK, unroll=2)
    def _(r):
        for j in range(8):
            b_v0[r, pl.ds(j * 16, 16)] = zero

    zbase = sid * ROWS_PER_TILE
    for k in range(ROWS_PER_TILE // CHUNK):
        pltpu.sync_copy(b_v0, acc.at[pl.ds(zbase + k * CHUNK, CHUNK)])
    plsc.subcore_barrier()

    ebase = wid * E_PER_TILE

    def fetch_idx(c, pk_s, sem_i):
        pltpu.async_copy(idx_hbm.at[pl.ds(ebase + c * CHUNK, CHUNK)],
                         pk_s, sem_i)

    def prep(c, pk_s, src_s, dst_s, a_v, b_v, sem_i, sem_a, sem_b):
        pltpu.make_async_copy(
            idx_hbm.at[pl.ds(ebase + c * CHUNK, CHUNK)], pk_s, sem_i).wait()

        @plsc.parallel_loop(0, CHUNK // 16)
        def _(k):
            s = pl.ds(k * 16, 16)
            w = pk_s[s]
            src_s[s] = w & 0xFFFF
            dst_s[s] = lax.shift_right_logical(w, 16)

        pltpu.async_copy(a_hbm.at[src_s], a_v, sem_a)
        pltpu.async_copy(b_hbm.at[pl.ds(ebase + c * CHUNK, CHUNK)], b_v, sem_b)

        @pl.when(c + 2 < CHUNKS_PER_TILE)
        def _():
            fetch_idx(c + 2, pk_s, sem_i)

    def process(c, src_s, dst_s, a_v, b_v, sem_a, sem_b):
        pltpu.make_async_copy(a_hbm.at[src_s], a_v, sem_a).wait()
        pltpu.make_async_copy(
            b_hbm.at[pl.ds(ebase + c * CHUNK, CHUNK)], b_v, sem_b).wait()

        @plsc.parallel_loop(0, CHUNK, unroll=2)
        def _(r):
            for j in range(8):
                s = pl.ds(j * 16, 16)
                a_v[r, s] = jnp.maximum(a_v[r, s] + b_v[r, s], 0.0)

        pltpu.sync_copy(a_v, acc.at[dst_s], add=True)

    fetch_idx(0, pk_s0, sem_i0)
    fetch_idx(1, pk_s1, sem_i1)
    prep(0, pk_s0, src_s0, dst_s0, a_v0, b_v0, sem_i0, sem_a0, sem_b0)
    n_groups = CHUNKS_PER_TILE // 2

    def group_body(g, carry):
        c0 = 2 * g
        prep(c0 + 1, pk_s1, src_s1, dst_s1, a_v1, b_v1,
             sem_i1, sem_a1, sem_b1)
        process(c0, src_s0, dst_s0, a_v0, b_v0, sem_a0, sem_b0)

        @pl.when(g < n_groups - 1)
        def _():
            prep(c0 + 2, pk_s0, src_s0, dst_s0, a_v0, b_v0,
                 sem_i0, sem_a0, sem_b0)

        process(c0 + 1, src_s1, dst_s1, a_v1, b_v1, sem_a1, sem_b1)
        return carry

    lax.fori_loop(0, n_groups, group_body, 0)
    plsc.subcore_barrier()

    obase = sid * ROWS_PER_TILE

    @pl.when(cid == 0)
    def _():
        pltpu.sync_copy(acc.at[pl.ds(obase, ROWS_PER_TILE)],
                        out0_hbm.at[pl.ds(obase, ROWS_PER_TILE)])

    @pl.when(cid == 1)
    def _():
        pltpu.sync_copy(acc.at[pl.ds(obase, ROWS_PER_TILE)],
                        out1_hbm.at[pl.ds(obase, ROWS_PER_TILE)])


def kernel(node_feats, edge_index, edge_feats, W_msg, b_msg, W_apply, b_apply):
    src = edge_index[0].astype(jnp.int32)
    dst = edge_index[1].astype(jnp.int32)
    pad = E_PAD - N_EDGES
    src = jnp.concatenate([src, jnp.zeros((pad,), jnp.int32)])
    dst = jnp.concatenate([dst, jnp.full((pad,), N_NODES, jnp.int32)])
    idx_packed = (dst << 16) | src
    ef_pad = jnp.concatenate(
        [edge_feats, jnp.zeros((pad, EF), jnp.float32)], axis=0
    )

    w1t = W_msg[:, :D].T
    w2t = W_msg[:, D:].T
    wa1t = W_apply[:, :D].T
    wa2t = W_apply[:, D:].T
    bm = b_msg.reshape(1, D)
    ba = b_apply.reshape(1, D)

    blk_a = 1000
    a = pl.pallas_call(
        _mm_bias_body,
        grid=(N_NODES // blk_a,),
        in_specs=[
            pl.BlockSpec((blk_a, D), lambda i: (i, 0)),
            pl.BlockSpec((D, D), lambda i: (0, 0)),
            pl.BlockSpec((1, D), lambda i: (0, 0)),
        ],
        out_specs=pl.BlockSpec((blk_a, D), lambda i: (i, 0)),
        out_shape=jax.ShapeDtypeStruct((N_NODES, D), jnp.float32),
    )(node_feats, w1t, bm)

    blk_b = 1024
    b = pl.pallas_call(
        _mm_body,
        grid=(E_PAD // blk_b,),
        in_specs=[
            pl.BlockSpec((blk_b, EF), lambda i: (i, 0)),
            pl.BlockSpec((EF, D), lambda i: (0, 0)),
        ],
        out_specs=pl.BlockSpec((blk_b, D), lambda i: (i, 0)),
        out_shape=jax.ShapeDtypeStruct((E_PAD, D), jnp.float32),
    )(ef_pad, w2t)

    part0, part1 = _sc_gather_scatter(a, b, idx_packed)
    p0 = part0[:N_NODES]
    p1 = part1[:N_NODES]

    out = pl.pallas_call(
        _apply_body,
        grid=(N_NODES // blk_a,),
        in_specs=[
            pl.BlockSpec((blk_a, D), lambda i: (i, 0)),
            pl.BlockSpec((blk_a, D), lambda i: (i, 0)),
            pl.BlockSpec((blk_a, D), lambda i: (i, 0)),
            pl.BlockSpec((D, D), lambda i: (0, 0)),
            pl.BlockSpec((D, D), lambda i: (0, 0)),
            pl.BlockSpec((1, D), lambda i: (0, 0)),
        ],
        out_specs=pl.BlockSpec((blk_a, D), lambda i: (i, 0)),
        out_shape=jax.ShapeDtypeStruct((N_NODES, D), jnp.float32),
    )(node_feats, p0, p1, wa1t, wa2t, ba)
    return out


# speedup vs baseline: 2.6861x; 1.1724x over previous
import functools

import jax
import jax.numpy as jnp
from jax import lax
from jax.experimental import pallas as pl
from jax.experimental.pallas import tpu as pltpu
from jax.experimental.pallas import tpu_sc as plsc

N_NODES = 10000
N_EDGES = 320000
D = 128
EF = 16

NUM_CORES = 2
NUM_SUBCORES = 16
NUM_TILES = NUM_CORES * NUM_SUBCORES
CHUNK = 64
CHUNKS_PER_TILE = 160
E_PER_TILE = CHUNK * CHUNKS_PER_TILE
E_PAD = NUM_TILES * E_PER_TILE
ACC_ROWS = 10240
ROWS_PER_TILE = ACC_ROWS // NUM_SUBCORES


def _mm_bias_body(x_ref, w_ref, b_ref, o_ref):
    o_ref[...] = (
        jnp.dot(x_ref[...], w_ref[...], preferred_element_type=jnp.float32)
        + b_ref[...]
    )


def _mm_body(x_ref, w_ref, o_ref):
    o_ref[...] = jnp.dot(x_ref[...], w_ref[...], preferred_element_type=jnp.float32)


def _apply_body(x_ref, p0_ref, p1_ref, w1_ref, w2_ref, b_ref, o_ref):
    m = p0_ref[...] + p1_ref[...]
    acc = jnp.dot(x_ref[...], w1_ref[...], preferred_element_type=jnp.float32)
    acc = acc + jnp.dot(m, w2_ref[...], preferred_element_type=jnp.float32)
    o_ref[...] = jnp.maximum(acc + b_ref[...], 0.0)


_sc_mesh = plsc.VectorSubcoreMesh(core_axis_name="c", subcore_axis_name="s")


@functools.partial(
    pl.kernel,
    out_type=[
        jax.ShapeDtypeStruct((ACC_ROWS, D), jnp.float32),
        jax.ShapeDtypeStruct((ACC_ROWS, D), jnp.float32),
    ],
    mesh=_sc_mesh,
    scratch_types=[
        pltpu.VMEM((CHUNK,), jnp.int32),
        pltpu.VMEM((CHUNK,), jnp.int32),
        pltpu.VMEM((CHUNK,), jnp.int32),
        pltpu.VMEM((CHUNK,), jnp.int32),
        pltpu.VMEM((CHUNK,), jnp.int32),
        pltpu.VMEM((CHUNK,), jnp.int32),
        pltpu.VMEM((CHUNK, D), jnp.float32),
        pltpu.VMEM((CHUNK, D), jnp.float32),
        pltpu.VMEM((CHUNK, D), jnp.float32),
        pltpu.VMEM((CHUNK, D), jnp.float32),
        pltpu.VMEM_SHARED((ACC_ROWS, D), jnp.float32),
        pltpu.SemaphoreType.DMA,
        pltpu.SemaphoreType.DMA,
        pltpu.SemaphoreType.DMA,
        pltpu.SemaphoreType.DMA,
        pltpu.SemaphoreType.DMA,
        pltpu.SemaphoreType.DMA,
    ],
)
def _sc_gather_scatter(a_hbm, b_hbm, idx_hbm, out0_hbm, out1_hbm,
                       pk_s0, pk_s1, src_s0, dst_s0, src_s1, dst_s1,
                       a_v0, b_v0, a_v1, b_v1, acc,
                       sem_i0, sem_i1, sem_a0, sem_b0, sem_a1, sem_b1):
    cid = lax.axis_index("c")
    sid = lax.axis_index("s")
    wid = cid * NUM_SUBCORES + sid

    zero = jnp.zeros((16,), jnp.float32)

    @plsc.parallel_loop(0, CHUNK, unroll=2)
    def _(r):
        for j in range(8):
            b_v0[r, pl.ds(j * 16, 16)] = zero

    zbase = sid * ROWS_PER_TILE
    for k in range(ROWS_PER_TILE // CHUNK):
        pltpu.sync_copy(b_v0, acc.at[pl.ds(zbase + k * CHUNK, CHUNK)])
    plsc.subcore_barrier()

    ebase = wid * E_PER_TILE

    def fetch_idx(c, pk_s, sem_i):
        pltpu.async_copy(idx_hbm.at[pl.ds(ebase + c * CHUNK, CHUNK)],
                         pk_s, sem_i)

    def prep(c, pk_s, src_s, dst_s, a_v, b_v, sem_i, sem_a, sem_b):
        pltpu.make_async_copy(
            idx_hbm.at[pl.ds(ebase + c * CHUNK, CHUNK)], pk_s, sem_i).wait()

        @plsc.parallel_loop(0, CHUNK // 16)
        def _(k):
            s = pl.ds(k * 16, 16)
            w = pk_s[s]
            src_s[s] = w & 0xFFFF
            dst_s[s] = lax.shift_right_logical(w, 16)

        pltpu.async_copy(a_hbm.at[src_s], a_v, sem_a)
        pltpu.async_copy(b_hbm.at[pl.ds(ebase + c * CHUNK, CHUNK)], b_v, sem_b)

        @pl.when(c + 2 < CHUNKS_PER_TILE)
        def _():
            fetch_idx(c + 2, pk_s, sem_i)

    def process(c, src_s, dst_s, a_v, b_v, sem_a, sem_b):
        pltpu.make_async_copy(a_hbm.at[src_s], a_v, sem_a).wait()
        pltpu.make_async_copy(
            b_hbm.at[pl.ds(ebase + c * CHUNK, CHUNK)], b_v, sem_b).wait()

        @plsc.parallel_loop(0, CHUNK, unroll=2)
        def _(r):
            for j in range(8):
                s = pl.ds(j * 16, 16)
                a_v[r, s] = jnp.maximum(a_v[r, s] + b_v[r, s], 0.0)

        pltpu.sync_copy(a_v, acc.at[dst_s], add=True)

    fetch_idx(0, pk_s0, sem_i0)
    fetch_idx(1, pk_s1, sem_i1)
    prep(0, pk_s0, src_s0, dst_s0, a_v0, b_v0, sem_i0, sem_a0, sem_b0)
    n_groups = CHUNKS_PER_TILE // 2

    def group_body(g, carry):
        c0 = 2 * g
        prep(c0 + 1, pk_s1, src_s1, dst_s1, a_v1, b_v1,
             sem_i1, sem_a1, sem_b1)
        process(c0, src_s0, dst_s0, a_v0, b_v0, sem_a0, sem_b0)

        @pl.when(g < n_groups - 1)
        def _():
            prep(c0 + 2, pk_s0, src_s0, dst_s0, a_v0, b_v0,
                 sem_i0, sem_a0, sem_b0)

        process(c0 + 1, src_s1, dst_s1, a_v1, b_v1, sem_a1, sem_b1)
        return carry

    lax.fori_loop(0, n_groups, group_body, 0)
    plsc.subcore_barrier()

    obase = sid * ROWS_PER_TILE

    @pl.when(cid == 0)
    def _():
        pltpu.sync_copy(acc.at[pl.ds(obase, ROWS_PER_TILE)],
                        out0_hbm.at[pl.ds(obase, ROWS_PER_TILE)])

    @pl.when(cid == 1)
    def _():
        pltpu.sync_copy(acc.at[pl.ds(obase, ROWS_PER_TILE)],
                        out1_hbm.at[pl.ds(obase, ROWS_PER_TILE)])


def kernel(node_feats, edge_index, edge_feats, W_msg, b_msg, W_apply, b_apply):
    src = edge_index[0].astype(jnp.int32)
    dst = edge_index[1].astype(jnp.int32)
    pad = E_PAD - N_EDGES
    src = jnp.concatenate([src, jnp.zeros((pad,), jnp.int32)])
    dummy = N_NODES + (jnp.arange(pad, dtype=jnp.int32) % (ACC_ROWS - N_NODES))
    dst = jnp.concatenate([dst, dummy])
    idx_packed = (dst << 16) | src

    w1t = W_msg[:, :D].T
    w2t = W_msg[:, D:].T
    wa1t = W_apply[:, :D].T
    wa2t = W_apply[:, D:].T
    bm = b_msg.reshape(1, D)
    ba = b_apply.reshape(1, D)

    blk_a = 1000
    a = pl.pallas_call(
        _mm_bias_body,
        grid=(N_NODES // blk_a,),
        in_specs=[
            pl.BlockSpec((blk_a, D), lambda i: (i, 0)),
            pl.BlockSpec((D, D), lambda i: (0, 0)),
            pl.BlockSpec((1, D), lambda i: (0, 0)),
        ],
        out_specs=pl.BlockSpec((blk_a, D), lambda i: (i, 0)),
        out_shape=jax.ShapeDtypeStruct((N_NODES, D), jnp.float32),
    )(node_feats, w1t, bm)

    blk_b = 3200
    b = pl.pallas_call(
        _mm_body,
        grid=(N_EDGES // blk_b,),
        in_specs=[
            pl.BlockSpec((blk_b, EF), lambda i: (i, 0)),
            pl.BlockSpec((EF, D), lambda i: (0, 0)),
        ],
        out_specs=pl.BlockSpec((blk_b, D), lambda i: (i, 0)),
        out_shape=jax.ShapeDtypeStruct((E_PAD, D), jnp.float32),
    )(edge_feats, w2t)

    part0, part1 = _sc_gather_scatter(a, b, idx_packed)
    p0 = part0[:N_NODES]
    p1 = part1[:N_NODES]

    out = pl.pallas_call(
        _apply_body,
        grid=(N_NODES // blk_a,),
        in_specs=[
            pl.BlockSpec((blk_a, D), lambda i: (i, 0)),
            pl.BlockSpec((blk_a, D), lambda i: (i, 0)),
            pl.BlockSpec((blk_a, D), lambda i: (i, 0)),
            pl.BlockSpec((D, D), lambda i: (0, 0)),
            pl.BlockSpec((D, D), lambda i: (0, 0)),
            pl.BlockSpec((1, D), lambda i: (0, 0)),
        ],
        out_specs=pl.BlockSpec((blk_a, D), lambda i: (i, 0)),
        out_shape=jax.ShapeDtypeStruct((N_NODES, D), jnp.float32),
    )(node_feats, p0, p1, wa1t, wa2t, ba)
    return out


# speedup vs baseline: 3.2228x; 1.1998x over previous
import functools

import jax
import jax.numpy as jnp
from jax import lax
from jax.experimental import pallas as pl
from jax.experimental.pallas import tpu as pltpu
from jax.experimental.pallas import tpu_sc as plsc

N_NODES = 10000
N_EDGES = 320000
D = 128
EF = 16

NUM_CORES = 2
NUM_SUBCORES = 16
NUM_TILES = NUM_CORES * NUM_SUBCORES
CHUNK = 64
CHUNKS_SC0 = 224
CHUNKS_SC1 = 96
TOTAL_CHUNKS = NUM_SUBCORES * (CHUNKS_SC0 + CHUNKS_SC1)
E_PAD = CHUNK * TOTAL_CHUNKS
ACC_ROWS = 10240
ROWS_PER_TILE = ACC_ROWS // NUM_SUBCORES


def _mm_bias_body(x_ref, w_ref, b_ref, o_ref):
    o_ref[...] = (
        jnp.dot(x_ref[...], w_ref[...], preferred_element_type=jnp.float32)
        + b_ref[...]
    )


def _mm_t_body(xt_ref, w_ref, o_ref):
    o_ref[...] = lax.dot_general(
        xt_ref[...], w_ref[...],
        dimension_numbers=(((0,), (0,)), ((), ())),
        preferred_element_type=jnp.float32,
    )


def _apply_body(x_ref, p0_ref, p1_ref, w1_ref, w2_ref, b_ref, o_ref):
    m = p0_ref[...] + p1_ref[...]
    acc = jnp.dot(x_ref[...], w1_ref[...], preferred_element_type=jnp.float32)
    acc = acc + jnp.dot(m, w2_ref[...], preferred_element_type=jnp.float32)
    o_ref[...] = jnp.maximum(acc + b_ref[...], 0.0)


_sc_mesh = plsc.VectorSubcoreMesh(core_axis_name="c", subcore_axis_name="s")


@functools.partial(
    pl.kernel,
    out_type=[
        jax.ShapeDtypeStruct((ACC_ROWS, D), jnp.float32),
        jax.ShapeDtypeStruct((ACC_ROWS, D), jnp.float32),
    ],
    mesh=_sc_mesh,
    scratch_types=[
        pltpu.VMEM((CHUNK,), jnp.int32),
        pltpu.VMEM((CHUNK,), jnp.int32),
        pltpu.VMEM((CHUNK,), jnp.int32),
        pltpu.VMEM((CHUNK,), jnp.int32),
        pltpu.VMEM((CHUNK,), jnp.int32),
        pltpu.VMEM((CHUNK,), jnp.int32),
        pltpu.VMEM((CHUNK, D), jnp.float32),
        pltpu.VMEM((CHUNK, D), jnp.float32),
        pltpu.VMEM((CHUNK, D), jnp.float32),
        pltpu.VMEM((CHUNK, D), jnp.float32),
        pltpu.VMEM_SHARED((ACC_ROWS, D), jnp.float32),
        pltpu.SemaphoreType.DMA,
        pltpu.SemaphoreType.DMA,
        pltpu.SemaphoreType.DMA,
        pltpu.SemaphoreType.DMA,
        pltpu.SemaphoreType.DMA,
        pltpu.SemaphoreType.DMA,
    ],
)
def _sc_gather_scatter(a_hbm, b_hbm, idx_hbm, out0_hbm, out1_hbm,
                       pk_s0, pk_s1, src_s0, dst_s0, src_s1, dst_s1,
                       a_v0, b_v0, a_v1, b_v1, acc,
                       sem_i0, sem_i1, sem_a0, sem_b0, sem_a1, sem_b1):
    cid = lax.axis_index("c")
    sid = lax.axis_index("s")
    wid = cid * NUM_SUBCORES + sid

    zero = jnp.zeros((16,), jnp.float32)

    @plsc.parallel_loop(0, CHUNK, unroll=2)
    def _(r):
        for j in range(8):
            b_v0[r, pl.ds(j * 16, 16)] = zero

    zbase = sid * ROWS_PER_TILE
    for k in range(ROWS_PER_TILE // CHUNK):
        pltpu.sync_copy(b_v0, acc.at[pl.ds(zbase + k * CHUNK, CHUNK)])
    plsc.subcore_barrier()

    n_chunks = jnp.where(cid == 0, CHUNKS_SC0, CHUNKS_SC1)
    cbase = jnp.where(cid == 0, sid * CHUNKS_SC0,
                      NUM_SUBCORES * CHUNKS_SC0 + sid * CHUNKS_SC1)
    ebase = cbase * CHUNK

    def fetch_idx(c, pk_s, sem_i):
        pltpu.async_copy(idx_hbm.at[pl.ds(ebase + c * CHUNK, CHUNK)],
                         pk_s, sem_i)

    def prep(c, pk_s, src_s, dst_s, a_v, b_v, sem_i, sem_a, sem_b):
        pltpu.make_async_copy(
            idx_hbm.at[pl.ds(ebase + c * CHUNK, CHUNK)], pk_s, sem_i).wait()

        @plsc.parallel_loop(0, CHUNK // 16)
        def _(k):
            s = pl.ds(k * 16, 16)
            w = pk_s[s]
            src_s[s] = w & 0xFFFF
            dst_s[s] = lax.shift_right_logical(w, 16)

        pltpu.async_copy(a_hbm.at[src_s], a_v, sem_a)
        pltpu.async_copy(b_hbm.at[pl.ds(ebase + c * CHUNK, CHUNK)], b_v, sem_b)

        @pl.when(c + 2 < n_chunks)
        def _():
            fetch_idx(c + 2, pk_s, sem_i)

    def process(c, src_s, dst_s, a_v, b_v, sem_a, sem_b):
        pltpu.make_async_copy(a_hbm.at[src_s], a_v, sem_a).wait()
        pltpu.make_async_copy(
            b_hbm.at[pl.ds(ebase + c * CHUNK, CHUNK)], b_v, sem_b).wait()

        @plsc.parallel_loop(0, CHUNK, unroll=2)
        def _(r):
            for j in range(8):
                s = pl.ds(j * 16, 16)
                a_v[r, s] = jnp.maximum(a_v[r, s] + b_v[r, s], 0.0)

        pltpu.sync_copy(a_v, acc.at[dst_s], add=True)

    fetch_idx(0, pk_s0, sem_i0)
    fetch_idx(1, pk_s1, sem_i1)
    prep(0, pk_s0, src_s0, dst_s0, a_v0, b_v0, sem_i0, sem_a0, sem_b0)
    n_groups = n_chunks // 2

    def group_body(g, carry):
        c0 = 2 * g
        prep(c0 + 1, pk_s1, src_s1, dst_s1, a_v1, b_v1,
             sem_i1, sem_a1, sem_b1)
        process(c0, src_s0, dst_s0, a_v0, b_v0, sem_a0, sem_b0)

        @pl.when(g < n_groups - 1)
        def _():
            prep(c0 + 2, pk_s0, src_s0, dst_s0, a_v0, b_v0,
                 sem_i0, sem_a0, sem_b0)

        process(c0 + 1, src_s1, dst_s1, a_v1, b_v1, sem_a1, sem_b1)
        return carry

    lax.fori_loop(0, n_groups, group_body, 0)
    plsc.subcore_barrier()

    obase = sid * ROWS_PER_TILE

    @pl.when(cid == 0)
    def _():
        pltpu.sync_copy(acc.at[pl.ds(obase, ROWS_PER_TILE)],
                        out0_hbm.at[pl.ds(obase, ROWS_PER_TILE)])

    @pl.when(cid == 1)
    def _():
        pltpu.sync_copy(acc.at[pl.ds(obase, ROWS_PER_TILE)],
                        out1_hbm.at[pl.ds(obase, ROWS_PER_TILE)])


def kernel(node_feats, edge_index, edge_feats, W_msg, b_msg, W_apply, b_apply):
    src = edge_index[0].astype(jnp.int32)
    dst = edge_index[1].astype(jnp.int32)
    pad = E_PAD - N_EDGES
    src = jnp.concatenate([src, jnp.zeros((pad,), jnp.int32)])
    dummy = N_NODES + (jnp.arange(pad, dtype=jnp.int32) % (ACC_ROWS - N_NODES))
    dst = jnp.concatenate([dst, dummy])
    idx_packed = (dst << 16) | src

    w1t = W_msg[:, :D].T
    w2t = W_msg[:, D:].T
    wa1t = W_apply[:, :D].T
    wa2t = W_apply[:, D:].T
    bm = b_msg.reshape(1, D)
    ba = b_apply.reshape(1, D)

    blk_a = 1000
    a = pl.pallas_call(
        _mm_bias_body,
        grid=(N_NODES // blk_a,),
        in_specs=[
            pl.BlockSpec((blk_a, D), lambda i: (i, 0)),
            pl.BlockSpec((D, D), lambda i: (0, 0)),
            pl.BlockSpec((1, D), lambda i: (0, 0)),
        ],
        out_specs=pl.BlockSpec((blk_a, D), lambda i: (i, 0)),
        out_shape=jax.ShapeDtypeStruct((N_NODES, D), jnp.float32),
    )(node_feats, w1t, bm)

    blk_b = 3200
    b = pl.pallas_call(
        _mm_t_body,
        grid=(N_EDGES // blk_b,),
        in_specs=[
            pl.BlockSpec((EF, blk_b), lambda i: (0, i)),
            pl.BlockSpec((EF, D), lambda i: (0, 0)),
        ],
        out_specs=pl.BlockSpec((blk_b, D), lambda i: (i, 0)),
        out_shape=jax.ShapeDtypeStruct((E_PAD, D), jnp.float32),
    )(edge_feats.T, w2t)

    part0, part1 = _sc_gather_scatter(a, b, idx_packed)
    p0 = part0[:N_NODES]
    p1 = part1[:N_NODES]

    out = pl.pallas_call(
        _apply_body,
        grid=(N_NODES // blk_a,),
        in_specs=[
            pl.BlockSpec((blk_a, D), lambda i: (i, 0)),
            pl.BlockSpec((blk_a, D), lambda i: (i, 0)),
            pl.BlockSpec((blk_a, D), lambda i: (i, 0)),
            pl.BlockSpec((D, D), lambda i: (0, 0)),
            pl.BlockSpec((D, D), lambda i: (0, 0)),
            pl.BlockSpec((1, D), lambda i: (0, 0)),
        ],
        out_specs=pl.BlockSpec((blk_a, D), lambda i: (i, 0)),
        out_shape=jax.ShapeDtypeStruct((N_NODES, D), jnp.float32),
    )(node_feats, p0, p1, wa1t, wa2t, ba)
    return out


# speedup vs baseline: 3.2635x; 1.0126x over previous
import functools

import jax
import jax.numpy as jnp
import numpy as np
from jax import lax
from jax.experimental import pallas as pl
from jax.experimental.pallas import tpu as pltpu
from jax.experimental.pallas import tpu_sc as plsc

N_NODES = 10000
N_EDGES = 320000
D = 128
EF = 16

NUM_CORES = 2
NUM_SUBCORES = 16
NUM_TILES = NUM_CORES * NUM_SUBCORES
CHUNK = 64
CHUNKS_SC0 = 246
CHUNKS_SC1 = 74
TOTAL_CHUNKS = NUM_SUBCORES * (CHUNKS_SC0 + CHUNKS_SC1)
E_PAD = CHUNK * TOTAL_CHUNKS

ACC_ROWS = 10240
ROWS_PER_TILE = ACC_ROWS // NUM_SUBCORES


def _mm_bias_body(x_ref, w_ref, b_ref, o_ref):
    o_ref[...] = (
        jnp.dot(x_ref[...], w_ref[...], preferred_element_type=jnp.float32)
        + b_ref[...]
    )


def _mm_t_body(xt_ref, w_ref, o_ref):
    o_ref[...] = lax.dot_general(
        xt_ref[...], w_ref[...],
        dimension_numbers=(((0,), (0,)), ((), ())),
        preferred_element_type=jnp.float32,
    )


def _apply_body(x_ref, p0_ref, p1_ref, w1_ref, w2_ref, b_ref, o_ref):
    m = p0_ref[...] + p1_ref[...]
    acc = jnp.dot(x_ref[...], w1_ref[...], preferred_element_type=jnp.float32)
    acc = acc + jnp.dot(m, w2_ref[...], preferred_element_type=jnp.float32)
    o_ref[...] = jnp.maximum(acc + b_ref[...], 0.0)


_sc_mesh = plsc.VectorSubcoreMesh(core_axis_name="c", subcore_axis_name="s")


@functools.partial(
    pl.kernel,
    out_type=[
        jax.ShapeDtypeStruct((ACC_ROWS, D), jnp.float32),
        jax.ShapeDtypeStruct((ACC_ROWS, D), jnp.float32),
    ],
    mesh=_sc_mesh,
    scratch_types=[
        pltpu.VMEM((CHUNK,), jnp.int32),
        pltpu.VMEM((CHUNK,), jnp.int32),
        pltpu.VMEM((CHUNK,), jnp.int32),
        pltpu.VMEM((CHUNK,), jnp.int32),
        pltpu.VMEM((CHUNK,), jnp.int32),
        pltpu.VMEM((CHUNK,), jnp.int32),
        pltpu.VMEM((CHUNK, D), jnp.float32),
        pltpu.VMEM((CHUNK, D), jnp.float32),
        pltpu.VMEM((CHUNK, D), jnp.float32),
        pltpu.VMEM((CHUNK, D), jnp.float32),
        pltpu.VMEM_SHARED((ACC_ROWS, D), jnp.float32),
        pltpu.SemaphoreType.DMA,
        pltpu.SemaphoreType.DMA,
        pltpu.SemaphoreType.DMA,
        pltpu.SemaphoreType.DMA,
        pltpu.SemaphoreType.DMA,
        pltpu.SemaphoreType.DMA,
    ],
)
def _sc_gather_scatter(a_hbm, b_hbm, idx_hbm, out0_hbm, out1_hbm,
                       pk_s0, pk_s1, src_s0, dst_s0, src_s1, dst_s1,
                       a_v0, b_v0, a_v1, b_v1, acc,
                       sem_i0, sem_i1, sem_a0, sem_b0, sem_a1, sem_b1):
    cid = lax.axis_index("c")
    sid = lax.axis_index("s")
    wid = cid * NUM_SUBCORES + sid

    zero = jnp.zeros((16,), jnp.float32)

    @plsc.parallel_loop(0, CHUNK, unroll=2)
    def _(r):
        for j in range(8):
            b_v0[r, pl.ds(j * 16, 16)] = zero

    zbase = sid * ROWS_PER_TILE
    for k in range(ROWS_PER_TILE // CHUNK):
        pltpu.sync_copy(b_v0, acc.at[pl.ds(zbase + k * CHUNK, CHUNK)])
    plsc.subcore_barrier()

    n_chunks = jnp.where(cid == 0, CHUNKS_SC0, CHUNKS_SC1)
    cbase = jnp.where(cid == 0, sid * CHUNKS_SC0,
                      NUM_SUBCORES * CHUNKS_SC0 + sid * CHUNKS_SC1)
    ebase = cbase * CHUNK

    def fetch_idx(c, pk_s, sem_i):
        pltpu.async_copy(idx_hbm.at[pl.ds(ebase + c * CHUNK, CHUNK)],
                         pk_s, sem_i)

    def prep(c, pk_s, src_s, dst_s, a_v, b_v, sem_i, sem_a, sem_b):
        pltpu.make_async_copy(
            idx_hbm.at[pl.ds(ebase + c * CHUNK, CHUNK)], pk_s, sem_i).wait()

        @plsc.parallel_loop(0, CHUNK // 16)
        def _(k):
            s = pl.ds(k * 16, 16)
            w = pk_s[s]
            src_s[s] = w & 0xFFFF
            dst_s[s] = lax.shift_right_logical(w, 16)

        pltpu.async_copy(a_hbm.at[src_s], a_v, sem_a)
        pltpu.async_copy(b_hbm.at[pl.ds(ebase + c * CHUNK, CHUNK)], b_v, sem_b)

        @pl.when(c + 2 < n_chunks)
        def _():
            fetch_idx(c + 2, pk_s, sem_i)

    def process(c, src_s, dst_s, a_v, b_v, sem_a, sem_b):
        pltpu.make_async_copy(a_hbm.at[src_s], a_v, sem_a).wait()
        pltpu.make_async_copy(
            b_hbm.at[pl.ds(ebase + c * CHUNK, CHUNK)], b_v, sem_b).wait()

        @plsc.parallel_loop(0, CHUNK, unroll=2)
        def _(r):
            for j in range(8):
                s = pl.ds(j * 16, 16)
                a_v[r, s] = jnp.maximum(a_v[r, s] + b_v[r, s], 0.0)

        pltpu.sync_copy(a_v, acc.at[dst_s], add=True)

    fetch_idx(0, pk_s0, sem_i0)
    fetch_idx(1, pk_s1, sem_i1)
    prep(0, pk_s0, src_s0, dst_s0, a_v0, b_v0, sem_i0, sem_a0, sem_b0)
    n_groups = n_chunks // 2

    def group_body(g, carry):
        c0 = 2 * g
        prep(c0 + 1, pk_s1, src_s1, dst_s1, a_v1, b_v1,
             sem_i1, sem_a1, sem_b1)
        process(c0, src_s0, dst_s0, a_v0, b_v0, sem_a0, sem_b0)

        @pl.when(g < n_groups - 1)
        def _():
            prep(c0 + 2, pk_s0, src_s0, dst_s0, a_v0, b_v0,
                 sem_i0, sem_a0, sem_b0)

        process(c0 + 1, src_s1, dst_s1, a_v1, b_v1, sem_a1, sem_b1)
        return carry

    lax.fori_loop(0, n_groups, group_body, 0)
    plsc.subcore_barrier()

    obase = sid * ROWS_PER_TILE

    @pl.when(cid == 0)
    def _():
        pltpu.sync_copy(acc.at[pl.ds(obase, ROWS_PER_TILE)],
                        out0_hbm.at[pl.ds(obase, ROWS_PER_TILE)])

    @pl.when(cid == 1)
    def _():
        pltpu.sync_copy(acc.at[pl.ds(obase, ROWS_PER_TILE)],
                        out1_hbm.at[pl.ds(obase, ROWS_PER_TILE)])


def kernel(node_feats, edge_index, edge_feats, W_msg, b_msg, W_apply, b_apply):
    src = edge_index[0].astype(jnp.int32)
    dst = edge_index[1].astype(jnp.int32)
    pad = E_PAD - N_EDGES
    src = jnp.concatenate([src, jnp.zeros((pad,), jnp.int32)])
    dummy = N_NODES + (jnp.arange(pad, dtype=jnp.int32) % (ACC_ROWS - N_NODES))
    dst = jnp.concatenate([dst, dummy])
    idx_packed = (dst << 16) | src

    w1t = W_msg[:, :D].T
    w2t = W_msg[:, D:].T
    wa1t = W_apply[:, :D].T
    wa2t = W_apply[:, D:].T
    bm = b_msg.reshape(1, D)
    ba = b_apply.reshape(1, D)

    blk_a = 1000
    a = pl.pallas_call(
        _mm_bias_body,
        grid=(N_NODES // blk_a,),
        in_specs=[
            pl.BlockSpec((blk_a, D), lambda i: (i, 0)),
            pl.BlockSpec((D, D), lambda i: (0, 0)),
            pl.BlockSpec((1, D), lambda i: (0, 0)),
        ],
        out_specs=pl.BlockSpec((blk_a, D), lambda i: (i, 0)),
        out_shape=jax.ShapeDtypeStruct((N_NODES, D), jnp.float32),
    )(node_feats, w1t, bm)

    blk_b = 3200
    b = pl.pallas_call(
        _mm_t_body,
        grid=(N_EDGES // blk_b,),
        in_specs=[
            pl.BlockSpec((EF, blk_b), lambda i: (0, i)),
            pl.BlockSpec((EF, D), lambda i: (0, 0)),
        ],
        out_specs=pl.BlockSpec((blk_b, D), lambda i: (i, 0)),
        out_shape=jax.ShapeDtypeStruct((E_PAD, D), jnp.float32),
    )(edge_feats.T, w2t)

    part0, part1 = _sc_gather_scatter(a, b, idx_packed)
    p0 = part0[:N_NODES]
    p1 = part1[:N_NODES]

    out = pl.pallas_call(
        _apply_body,
        grid=(N_NODES // blk_a,),
        in_specs=[
            pl.BlockSpec((blk_a, D), lambda i: (i, 0)),
            pl.BlockSpec((blk_a, D), lambda i: (i, 0)),
            pl.BlockSpec((blk_a, D), lambda i: (i, 0)),
            pl.BlockSpec((D, D), lambda i: (0, 0)),
            pl.BlockSpec((D, D), lambda i: (0, 0)),
            pl.BlockSpec((1, D), lambda i: (0, 0)),
        ],
        out_specs=pl.BlockSpec((blk_a, D), lambda i: (i, 0)),
        out_shape=jax.ShapeDtypeStruct((N_NODES, D), jnp.float32),
    )(node_feats, p0, p1, wa1t, wa2t, ba)
    return out
